# baseline (speedup 1.0000x reference)
import functools

import jax
import jax.numpy as jnp
from jax import lax
from jax.experimental import pallas as pl
from jax.experimental.pallas import tpu as pltpu
from jax.experimental.pallas import tpu_sc as plsc

_BATCH = 4096
_SEQ = 50
_D = 64
_NW = 32
_BL = _BATCH // _NW
_NBUF = 5
_LOOKAHEAD = 3
_TP = 129

_mesh = plsc.VectorSubcoreMesh(core_axis_name="c", subcore_axis_name="s")


@functools.partial(
    pl.kernel,
    mesh=_mesh,
    out_type=jax.ShapeDtypeStruct((_SEQ, _D // 8, _NW, 8, _BL), jnp.float32),
    scratch_types=[
        pltpu.VMEM((_BL // 8, 8, 128), jnp.int32),
        pltpu.VMEM((64, _TP), jnp.int32),
        pltpu.VMEM((_NBUF, _BL, _D), jnp.float32),
        pltpu.VMEM((_NBUF, _D // 8, 8, _TP), jnp.float32),
        pltpu.SemaphoreType.DMA((_NBUF,)),
        pltpu.SemaphoreType.DMA((_NBUF,)),
    ],
    compiler_params=pltpu.CompilerParams(
        use_tc_tiling_on_sc=False, needs_layout_passes=False),
)
def _embed_lookup(idx_hbm, table_hbm, out_hbm, idx_v, idx_t, g_v, t_v,
                  gsem, osem):
    wid = lax.axis_index("s") * 2 + lax.axis_index("c")
    pltpu.sync_copy(idx_hbm.at[pl.ds(wid * (_BL // 8), _BL // 8)], idx_v)

    lanes = lax.iota(jnp.int32, 16)
    dt_vecs = [(lanes + 16 * q) // 8 for q in range(_D // 16)]
    dr_vecs = [(lanes + 16 * q) % 8 for q in range(_D // 16)]

    _QOFF = (0, 16, 32, 48)
    s_vecs = [lanes + o for o in _QOFF]

    @plsc.parallel_loop(0, _BL, unroll=4)
    def _(r):
        rsplat = lanes * 0 + r
        for o, svec in zip(_QOFF, s_vecs):
            v = idx_v[r // 8, r % 8, pl.ds(o, 16)]
            plsc.store_scatter(idx_t, [svec, rsplat], v + v)

    def idx_row(s):
        return idx_t.at[s, pl.ds(0, _BL)]

    for s in range(_LOOKAHEAD):
        pltpu.async_copy(table_hbm.at[idx_row(s)], g_v.at[s], gsem.at[s])

    def outer(g, _):
        for b in range(_NBUF):
            s = g * _NBUF + b
            nb = (b + _LOOKAHEAD) % _NBUF
            pltpu.make_async_copy(
                table_hbm.at[idx_row(b)], g_v.at[b], gsem.at[b]).wait()

            @pl.when(s >= _NBUF)
            def _():
                pltpu.make_async_copy(
                    t_v.at[b, :, :, pl.ds(0, _BL)],
                    out_hbm.at[0, :, wid], osem.at[b]).wait()

            gb = g_v.at[b]
            tb = t_v.at[b]

            @plsc.parallel_loop(0, _BL, unroll=4)
            def _(r):
                rsplat = lanes * 0 + r
                for q in range(_D // 16):
                    v = gb[r, pl.ds(16 * q, 16)]
                    plsc.store_scatter(
                        tb, [dt_vecs[q], dr_vecs[q], rsplat], v)

            pltpu.async_copy(
                t_v.at[b, :, :, pl.ds(0, _BL)], out_hbm.at[s, :, wid],
                osem.at[b])

            @pl.when(s + _LOOKAHEAD < _SEQ)
            def _():
                pltpu.async_copy(
                    table_hbm.at[idx_row(s + _LOOKAHEAD)], g_v.at[nb],
                    gsem.at[nb])

        return 0

    lax.fori_loop(0, _SEQ // _NBUF, outer, 0)
    for b in range(_NBUF):
        pltpu.make_async_copy(
            t_v.at[b, :, :, pl.ds(0, _BL)], out_hbm.at[0, :, wid],
            osem.at[b]).wait()


def _pad_rows_body(t_ref, o_ref):
    o_ref[:, : _D] = t_ref[...]


_pad_rows = pl.pallas_call(
    _pad_rows_body,
    grid=(100,),
    in_specs=[pl.BlockSpec((1000, _D), lambda i: (i, 0))],
    out_specs=pl.BlockSpec((1000, 128), lambda i: (i, 0)),
    out_shape=jax.ShapeDtypeStruct((100000, 128), jnp.float32),
)


def kernel(input_seq, embed_table):
    idx = jnp.pad(input_seq.astype(jnp.int32), ((0, 0), (0, 128 - _SEQ)))
    idx = idx.reshape(_BATCH // 8, 8, 128)
    table2 = _pad_rows(embed_table).reshape(-1, _D)
    out5 = _embed_lookup(idx, table2)
    out = out5.transpose(2, 4, 0, 1, 3)
    return out.reshape(_BATCH, _SEQ, _D)


# speedup vs baseline: 1.5003x; 1.5003x over previous
import functools

import jax
import jax.numpy as jnp
from jax import lax
from jax.experimental import pallas as pl
from jax.experimental.pallas import tpu as pltpu
from jax.experimental.pallas import tpu_sc as plsc

_BATCH = 4096
_SEQ = 50
_D = 64
_NW = 32
_BL = _BATCH // _NW
_NBUF = 5
_LOOKAHEAD = 3
_TP = 129

_mesh = plsc.VectorSubcoreMesh(core_axis_name="c", subcore_axis_name="s")


@functools.partial(
    pl.kernel,
    mesh=_mesh,
    out_type=jax.ShapeDtypeStruct((_SEQ, _D // 8, _NW, 8, _BL), jnp.float32),
    scratch_types=[
        pltpu.VMEM((_BL // 8, 8, 128), jnp.int32),
        pltpu.VMEM((64, _TP), jnp.int32),
        pltpu.VMEM((_NBUF, _BL, _D), jnp.float32),
        pltpu.VMEM((_NBUF, _D // 8, 8, _TP), jnp.float32),
        pltpu.SemaphoreType.DMA((_NBUF,)),
        pltpu.SemaphoreType.DMA((_NBUF,)),
    ],
    compiler_params=pltpu.CompilerParams(
        use_tc_tiling_on_sc=False, needs_layout_passes=False),
)
def _embed_lookup(idx_hbm, table_hbm, out_hbm, idx_v, idx_t, g_v, t_v,
                  gsem, osem):
    wid = lax.axis_index("s") * 2 + lax.axis_index("c")
    pltpu.sync_copy(idx_hbm.at[pl.ds(wid * (_BL // 8), _BL // 8)], idx_v)

    lanes = lax.iota(jnp.int32, 16)
    dt_vecs = [(lanes + 16 * q) // 8 for q in range(_D // 16)]
    dr_vecs = [(lanes + 16 * q) % 8 for q in range(_D // 16)]

    _QOFF = (0, 16, 32, 48)
    s_vecs = [lanes + o for o in _QOFF]

    @plsc.parallel_loop(0, _BL, unroll=4)
    def _(r):
        rsplat = lanes * 0 + r
        for o, svec in zip(_QOFF, s_vecs):
            v = idx_v[r // 8, r % 8, pl.ds(o, 16)]
            plsc.store_scatter(idx_t, [svec, rsplat], v + v)

    def idx_row(s):
        return idx_t.at[s, pl.ds(0, _BL)]

    for s in range(_LOOKAHEAD):
        pltpu.async_copy(table_hbm.at[idx_row(s)], g_v.at[s], gsem.at[s])

    def outer(g, _):
        for b in range(_NBUF):
            s = g * _NBUF + b
            nb = (b + _LOOKAHEAD) % _NBUF
            pltpu.make_async_copy(
                table_hbm.at[idx_row(b)], g_v.at[b], gsem.at[b]).wait()

            @pl.when(s >= _NBUF)
            def _():
                pltpu.make_async_copy(
                    t_v.at[b, :, :, pl.ds(0, _BL)],
                    out_hbm.at[0, :, wid], osem.at[b]).wait()

            gb = g_v.at[b]
            tb = t_v.at[b]

            @plsc.parallel_loop(0, _BL, unroll=4)
            def _(r):
                rsplat = lanes * 0 + r
                for q in range(_D // 16):
                    v = gb[r, pl.ds(16 * q, 16)]
                    plsc.store_scatter(
                        tb, [dt_vecs[q], dr_vecs[q], rsplat], v)

            pltpu.async_copy(
                t_v.at[b, :, :, pl.ds(0, _BL)], out_hbm.at[s, :, wid],
                osem.at[b])

            @pl.when(s + _LOOKAHEAD < _SEQ)
            def _():
                pltpu.async_copy(
                    table_hbm.at[idx_row(s + _LOOKAHEAD)], g_v.at[nb],
                    gsem.at[nb])

        return 0

    lax.fori_loop(0, _SEQ // _NBUF, outer, 0)
    for b in range(_NBUF):
        pltpu.make_async_copy(
            t_v.at[b, :, :, pl.ds(0, _BL)], out_hbm.at[0, :, wid],
            osem.at[b]).wait()


def kernel(input_seq, embed_table):
    idx = jnp.pad(input_seq.astype(jnp.int32), ((0, 0), (0, 128 - _SEQ)))
    idx = idx.reshape(_BATCH // 8, 8, 128)
    table2 = jnp.concatenate(
        [embed_table, jnp.zeros((100000, 64), jnp.float32)], axis=1
    ).reshape(-1, _D)
    out5 = _embed_lookup(idx, table2)
    out = out5.transpose(2, 4, 0, 1, 3)
    return out.reshape(_BATCH, _SEQ, _D)


# speedup vs baseline: 1.5072x; 1.0046x over previous
import functools

import jax
import jax.numpy as jnp
from jax import lax
from jax.experimental import pallas as pl
from jax.experimental.pallas import tpu as pltpu
from jax.experimental.pallas import tpu_sc as plsc

_BATCH = 4096
_SEQ = 50
_D = 64
_NW = 32
_BL = _BATCH // _NW
_NBUF = 5
_LOOKAHEAD = 3
_TP = 129

_mesh = plsc.VectorSubcoreMesh(core_axis_name="c", subcore_axis_name="s")


@functools.partial(
    pl.kernel,
    mesh=_mesh,
    out_type=jax.ShapeDtypeStruct((_SEQ, _D // 8, _NW, 8, _BL), jnp.float32),
    scratch_types=[
        pltpu.VMEM((_BL // 8, 8, 128), jnp.int32),
        pltpu.VMEM((64, _TP), jnp.int32),
        pltpu.VMEM((_NBUF, _BL, _D), jnp.float32),
        pltpu.VMEM((_NBUF, _D // 8, 8, _TP), jnp.float32),
        pltpu.SemaphoreType.DMA((_NBUF,)),
        pltpu.SemaphoreType.DMA((_NBUF,)),
    ],
    compiler_params=pltpu.CompilerParams(
        use_tc_tiling_on_sc=False, needs_layout_passes=False),
)
def _embed_lookup(idx_hbm, table_hbm, out_hbm, idx_v, idx_t, g_v, t_v,
                  gsem, osem):
    wid = lax.axis_index("s") * 2 + lax.axis_index("c")
    pltpu.sync_copy(idx_hbm.at[pl.ds(wid * (_BL // 8), _BL // 8)], idx_v)

    lanes = lax.iota(jnp.int32, 16)
    dt_vecs = [(lanes + 16 * q) // 8 for q in range(_D // 16)]
    dr_vecs = [(lanes + 16 * q) % 8 for q in range(_D // 16)]

    _QOFF = (0, 16, 32, 48)
    s_vecs = [lanes + o for o in _QOFF]

    @plsc.parallel_loop(0, _BL, unroll=4)
    def _(r):
        rsplat = lanes * 0 + r
        for o, svec in zip(_QOFF, s_vecs):
            v = idx_v[r // 8, r % 8, pl.ds(o, 16)]
            plsc.store_scatter(idx_t, [svec, rsplat], v + v)

    def idx_row(s):
        return idx_t.at[s, pl.ds(0, _BL)]

    for s in range(_LOOKAHEAD):
        pltpu.async_copy(table_hbm.at[idx_row(s)], g_v.at[s], gsem.at[s])

    def outer(g, _):
        for b in range(_NBUF):
            s = g * _NBUF + b
            nb = (b + _LOOKAHEAD) % _NBUF
            pltpu.make_async_copy(
                table_hbm.at[idx_row(b)], g_v.at[b], gsem.at[b]).wait()

            @pl.when(s >= _NBUF)
            def _():
                pltpu.make_async_copy(
                    t_v.at[b, :, :, pl.ds(0, _BL)],
                    out_hbm.at[0, :, wid], osem.at[b]).wait()

            gb = g_v.at[b]
            tb = t_v.at[b]

            @plsc.parallel_loop(0, _BL, unroll=4)
            def _(r):
                rsplat = lanes * 0 + r
                for q in range(_D // 16):
                    v = gb[r, pl.ds(16 * q, 16)]
                    plsc.store_scatter(
                        tb, [dt_vecs[q], dr_vecs[q], rsplat], v)

            pltpu.async_copy(
                t_v.at[b, :, :, pl.ds(0, _BL)], out_hbm.at[s, :, wid],
                osem.at[b])

            @pl.when(s + _LOOKAHEAD < _SEQ)
            def _():
                pltpu.async_copy(
                    table_hbm.at[idx_row(s + _LOOKAHEAD)], g_v.at[nb],
                    gsem.at[nb])

        return 0

    lax.fori_loop(0, _SEQ // _NBUF, outer, 0)
    for b in range(_NBUF):
        pltpu.make_async_copy(
            t_v.at[b, :, :, pl.ds(0, _BL)], out_hbm.at[0, :, wid],
            osem.at[b]).wait()


def kernel(input_seq, embed_table):
    idx = jnp.pad(input_seq.astype(jnp.int32), ((0, 0), (0, 128 - _SEQ)))
    idx = idx.reshape(_BATCH // 8, 8, 128)
    table2 = jnp.pad(embed_table, ((0, 0), (0, 64))).reshape(-1, _D)
    out5 = _embed_lookup(idx, table2)
    out = out5.transpose(2, 4, 0, 1, 3)
    return out.reshape(_BATCH, _SEQ, _D)
